# baseline (speedup 1.0000x reference)
import jax
import jax.numpy as jnp
from jax.experimental import pallas as pl
from jax.experimental.pallas import tpu as pltpu

_B, _N, _D, _K = 32, 1024, 64, 1024
_TB = 2
_TN = _TB * _N
_NT = _B // _TB


def _vq_body(x_ref, e_ref, idx_ref, q_ref, en_ref, wc_ref, dist_ref):
    i = pl.program_id(0)

    @pl.when(i == 0)
    def _init():
        e = e_ref[...]
        en_ref[...] = e / jnp.clip(
            jnp.sqrt(jnp.sum(e * e, axis=1, keepdims=True)), 1e-12)
        col = jax.lax.broadcasted_iota(jnp.int32, (_K, 64), 1)
        kio = jax.lax.broadcasted_iota(jnp.int32, (_K, 64), 0).astype(
            jnp.float32)
        extra = jnp.where(col == 0, kio, jnp.where(col == 1, 1.0, 0.0))
        wc_ref[...] = jnp.concatenate([e_ref[...], extra], axis=1)

    xt = x_ref[...].reshape(_TN, _D)
    xn = xt / jnp.clip(
        jnp.sqrt(jnp.sum(xt * xt, axis=1, keepdims=True)), 1e-12)
    dist_ref[i % 2] = jax.lax.dot_general(
        xn, en_ref[...], (((1,), (1,)), ((), ())),
        preferred_element_type=jnp.float32)

    dp = dist_ref[(i + 1) % 2]
    m = jnp.max(dp, axis=1, keepdims=True)
    eqf = jnp.where(dp == m, 1.0, 0.0)
    sums = jax.lax.dot_general(
        eqf, wc_ref[...], (((1,), (0,)), ((), ())),
        preferred_element_type=jnp.float32)
    tie = jnp.max(sums[:, 65]) > 1.5

    @pl.when(jnp.logical_not(tie))
    def _fast():
        idx_ref[...] = sums[:, 64].astype(jnp.int32).reshape(1, 1, _TN)
        q_ref[...] = sums[:, :64].reshape(_TB, _N, _D)

    @pl.when(tie)
    def _slow():
        ids = jax.lax.broadcasted_iota(jnp.int32, dp.shape, 1)
        idxt = jnp.min(jnp.where(dp == m, ids, _K), axis=1)
        oh = (ids == idxt[:, None]).astype(jnp.float32)
        qv = jax.lax.dot_general(
            oh, e_ref[...], (((1,), (0,)), ((), ())),
            preferred_element_type=jnp.float32,
            precision=jax.lax.Precision.HIGHEST)
        idx_ref[...] = idxt.reshape(1, 1, _TN)
        q_ref[...] = qv.reshape(_TB, _N, _D)


def kernel(x, embed):
    e2 = embed[0]
    idx_out, q_out = pl.pallas_call(
        _vq_body,
        grid=(_NT + 1,),
        in_specs=[
            pl.BlockSpec((_TB, _N, _D),
                         lambda i: (jnp.minimum(i, _NT - 1), 0, 0)),
            pl.BlockSpec((_K, _D), lambda i: (0, 0)),
        ],
        out_specs=[
            pl.BlockSpec((1, 1, _TN),
                         lambda i: (jnp.maximum(i - 1, 0), 0, 0)),
            pl.BlockSpec((_TB, _N, _D),
                         lambda i: (jnp.maximum(i - 1, 0), 0, 0)),
        ],
        out_shape=[
            jax.ShapeDtypeStruct((_NT, 1, _TN), jnp.int32),
            jax.ShapeDtypeStruct((_B, _N, _D), jnp.float32),
        ],
        scratch_shapes=[
            pltpu.VMEM((_K, _D), jnp.float32),
            pltpu.VMEM((_K, 128), jnp.float32),
            pltpu.VMEM((2, _TN, _K), jnp.float32),
        ],
    )(x, e2)
    return q_out, idx_out.reshape(_B, _N)


# speedup vs baseline: 1.1503x; 1.1503x over previous
import jax
import jax.numpy as jnp
from jax.experimental import pallas as pl
from jax.experimental.pallas import tpu as pltpu

_B, _N, _D, _K = 32, 1024, 64, 1024
_BB = 8
_GRID = _B // _BB


def _vq_body(x_ref, e_ref, idx_ref, q_ref, en_ref, w1_ref, w2_ref):
    @pl.when(pl.program_id(0) == 0)
    def _init():
        e = e_ref[...]
        en_ref[...] = e / jnp.clip(
            jnp.sqrt(jnp.sum(e * e, axis=1, keepdims=True)), 1e-12)
        ehi = e.astype(jnp.bfloat16).astype(jnp.float32)
        col = jax.lax.broadcasted_iota(jnp.int32, (_K, 64), 1)
        kio = jax.lax.broadcasted_iota(jnp.int32, (_K, 64), 0)
        hi = (kio // 32).astype(jnp.float32)
        lo = (kio % 32).astype(jnp.float32)
        extra = jnp.where(col == 0, hi,
                          jnp.where(col == 1, lo,
                                    jnp.where(col == 2, 1.0, 0.0)))
        w1_ref[...] = jnp.concatenate([ehi, extra], axis=1)
        w2_ref[...] = e - ehi

    def _tile(t):
        xt = x_ref[t]
        xn = xt / jnp.clip(
            jnp.sqrt(jnp.sum(xt * xt, axis=1, keepdims=True)), 1e-12)
        dist = jax.lax.dot_general(
            xn, en_ref[...], (((1,), (1,)), ((), ())),
            preferred_element_type=jnp.float32)
        m = jnp.max(dist, axis=1, keepdims=True)
        eqf = jnp.where(dist == m, 1.0, 0.0)
        s1 = jax.lax.dot_general(
            eqf, w1_ref[...], (((1,), (0,)), ((), ())),
            preferred_element_type=jnp.float32)
        s2 = jax.lax.dot_general(
            eqf, w2_ref[...], (((1,), (0,)), ((), ())),
            preferred_element_type=jnp.float32)
        tie = jnp.max(s1[:, 66]) > 1.5

        @pl.when(jnp.logical_not(tie))
        def _fast():
            idx_ref[pl.ds(t, 1), :] = (
                32 * s1[:, 64].astype(jnp.int32)
                + s1[:, 65].astype(jnp.int32))[None, :]
            q_ref[t] = s1[:, :64] + s2

        @pl.when(tie)
        def _slow():
            ids = jax.lax.broadcasted_iota(jnp.int32, dist.shape, 1)
            idxt = jnp.min(jnp.where(dist == m, ids, _K), axis=1)
            oh = (ids == idxt[:, None]).astype(jnp.float32)
            qv = jax.lax.dot_general(
                oh, e_ref[...], (((1,), (0,)), ((), ())),
                preferred_element_type=jnp.float32,
                precision=jax.lax.Precision.HIGHEST)
            idx_ref[pl.ds(t, 1), :] = idxt[None, :]
            q_ref[t] = qv

    def _pair(j, carry):
        _tile(2 * j)
        _tile(2 * j + 1)
        return carry

    jax.lax.fori_loop(0, _BB // 2, _pair, 0)


def kernel(x, embed):
    e2 = embed[0]
    idx_out, q_out = pl.pallas_call(
        _vq_body,
        grid=(_GRID,),
        in_specs=[
            pl.BlockSpec((_BB, _N, _D), lambda i: (i, 0, 0)),
            pl.BlockSpec((_K, _D), lambda i: (0, 0)),
        ],
        out_specs=[
            pl.BlockSpec((_BB, _N), lambda i: (i, 0)),
            pl.BlockSpec((_BB, _N, _D), lambda i: (i, 0, 0)),
        ],
        out_shape=[
            jax.ShapeDtypeStruct((_B, _N), jnp.int32),
            jax.ShapeDtypeStruct((_B, _N, _D), jnp.float32),
        ],
        scratch_shapes=[
            pltpu.VMEM((_K, _D), jnp.float32),
            pltpu.VMEM((_K, 128), jnp.float32),
            pltpu.VMEM((_K, _D), jnp.float32),
        ],
    )(x, e2)
    return q_out, idx_out


# speedup vs baseline: 1.2053x; 1.0478x over previous
import jax
import jax.numpy as jnp
from jax.experimental import pallas as pl
from jax.experimental.pallas import tpu as pltpu

_B, _N, _D, _K = 32, 1024, 64, 1024
_BB = 8
_GRID = _B // _BB


def _vq_body(x_ref, e_ref, idx_ref, q_ref, en_ref, w1_ref, w2_ref):
    @pl.when(pl.program_id(0) == 0)
    def _init():
        e = e_ref[...]
        en_ref[...] = e / jnp.clip(
            jnp.sqrt(jnp.sum(e * e, axis=1, keepdims=True)), 1e-12)
        ehi = e.astype(jnp.bfloat16).astype(jnp.float32)
        col = jax.lax.broadcasted_iota(jnp.int32, (_K, 64), 1)
        kio = jax.lax.broadcasted_iota(jnp.int32, (_K, 64), 0)
        hi = (kio // 32).astype(jnp.float32)
        lo = (kio % 32).astype(jnp.float32)
        extra = jnp.where(col == 0, hi,
                          jnp.where(col == 1, lo,
                                    jnp.where(col == 2, 1.0, 0.0)))
        w1_ref[...] = jnp.concatenate([ehi, extra], axis=1)
        w2_ref[...] = e - ehi

    def _compute(t):
        xt = x_ref[t]
        xn = xt / jnp.clip(
            jnp.sqrt(jnp.sum(xt * xt, axis=1, keepdims=True)), 1e-12)
        dist = jax.lax.dot_general(
            xn, en_ref[...], (((1,), (1,)), ((), ())),
            preferred_element_type=jnp.float32)
        m = jnp.max(dist, axis=1, keepdims=True)
        eqf = jnp.where(dist == m, 1.0, 0.0)
        s1 = jax.lax.dot_general(
            eqf, w1_ref[...], (((1,), (0,)), ((), ())),
            preferred_element_type=jnp.float32)
        s2 = jax.lax.dot_general(
            eqf, w2_ref[...], (((1,), (0,)), ((), ())),
            preferred_element_type=jnp.float32)
        return dist, m, s1, s2

    def _store(t, dist, m, s1, s2):
        tie = jnp.max(s1[:, 66]) > 1.5

        @pl.when(jnp.logical_not(tie))
        def _fast():
            idx_ref[pl.ds(t, 1), :] = (
                32 * s1[:, 64].astype(jnp.int32)
                + s1[:, 65].astype(jnp.int32))[None, :]
            q_ref[t] = s1[:, :64] + s2

        @pl.when(tie)
        def _slow():
            ids = jax.lax.broadcasted_iota(jnp.int32, dist.shape, 1)
            idxt = jnp.min(jnp.where(dist == m, ids, _K), axis=1)
            oh = (ids == idxt[:, None]).astype(jnp.float32)
            qv = jax.lax.dot_general(
                oh, e_ref[...], (((1,), (0,)), ((), ())),
                preferred_element_type=jnp.float32,
                precision=jax.lax.Precision.HIGHEST)
            idx_ref[pl.ds(t, 1), :] = idxt[None, :]
            q_ref[t] = qv

    def _pair(j, carry):
        ra = _compute(2 * j)
        rb = _compute(2 * j + 1)
        _store(2 * j, *ra)
        _store(2 * j + 1, *rb)
        return carry

    jax.lax.fori_loop(0, _BB // 2, _pair, 0)


def kernel(x, embed):
    e2 = embed[0]
    idx_out, q_out = pl.pallas_call(
        _vq_body,
        grid=(_GRID,),
        in_specs=[
            pl.BlockSpec((_BB, _N, _D), lambda i: (i, 0, 0)),
            pl.BlockSpec((_K, _D), lambda i: (0, 0)),
        ],
        out_specs=[
            pl.BlockSpec((_BB, _N), lambda i: (i, 0)),
            pl.BlockSpec((_BB, _N, _D), lambda i: (i, 0, 0)),
        ],
        out_shape=[
            jax.ShapeDtypeStruct((_B, _N), jnp.int32),
            jax.ShapeDtypeStruct((_B, _N, _D), jnp.float32),
        ],
        scratch_shapes=[
            pltpu.VMEM((_K, _D), jnp.float32),
            pltpu.VMEM((_K, 128), jnp.float32),
            pltpu.VMEM((_K, _D), jnp.float32),
        ],
    )(x, e2)
    return q_out, idx_out
